# baseline (speedup 1.0000x reference)
import functools

import jax
import jax.numpy as jnp
from jax import lax
from jax.experimental import pallas as pl
from jax.experimental.pallas import tpu as pltpu
from jax.experimental.pallas import tpu_sc as plsc

N = 10000
E = 320000
R = 5
NG = 500
PG = 20
H = 32
NC, NS = 2, 16
NW = NC * NS
CH = 128
NCHUNK = 80
EPT = CH * NCHUNK
EPAD = EPT * NW
AROWS = R * N + 48
RPT = AROWS // NS
N4 = N // 4
A4 = AROWS // 4


@functools.lru_cache(maxsize=None)
def _sc_mesh():
    return plsc.VectorSubcoreMesh(
        core_axis_name="c", subcore_axis_name="s",
        num_cores=NC, num_subcores=NS)


GRP = 2


def _sc_edge_body(table_hbm, pidx_hbm, zeros_hbm, out_hbm,
                  pidx, ga0, ga1, gb0, gb1, sa0, sa1, sb0, sb1,
                  da0, da1, db0, db1,
                  acc, semga, semgb, semsa, semsb):
    gidx = [[ga0, ga1], [gb0, gb1]]
    sidx = [[sa0, sa1], [sb0, sb1]]
    data = [[da0, da1], [db0, db1]]
    gsem = [semga, semgb]
    ssem = [semsa, semsb]
    cid = lax.axis_index("c")
    sid = lax.axis_index("s")
    wid = cid * NS + sid
    pltpu.sync_copy(zeros_hbm, acc.at[pl.ds(sid * RPT, RPT)])
    plsc.subcore_barrier()
    pltpu.sync_copy(pidx_hbm.at[wid], pidx)

    def unpack(j, grp, k):
        for i in range(CH // 16):
            pr = pidx[j, pl.ds(i * 16, 16)]
            gidx[grp][k][pl.ds(i * 16, 16)] = pr & 0xFFFF
            sidx[grp][k][pl.ds(i * 16, 16)] = lax.shift_right_logical(pr, 16)

    def drain(buf, sem):
        pltpu.make_async_copy(table_hbm.at[pl.ds(0, CH)], buf, sem).wait()

    for grp in range(2):
        for k in range(GRP):
            unpack(grp * GRP + k, grp, k)
            pltpu.async_copy(table_hbm.at[gidx[grp][k]], data[grp][k],
                             gsem[grp])

    def step(j4, carry):
        base = j4 * 2 * GRP
        for grp in range(2):
            for k in range(GRP):
                drain(data[grp][k], gsem[grp])
            for k in range(GRP):
                pltpu.async_copy(data[grp][k], acc.at[sidx[grp][k]],
                                 ssem[grp], add=True)
        for grp in range(2):
            for k in range(GRP):
                drain(data[grp][k], ssem[grp])
            for k in range(GRP):
                jn = jnp.minimum(base + 2 * GRP + grp * GRP + k, NCHUNK - 1)
                unpack(jn, grp, k)
                pltpu.async_copy(table_hbm.at[gidx[grp][k]], data[grp][k],
                                 gsem[grp])
        return carry

    lax.fori_loop(0, NCHUNK // (2 * GRP), step, 0, unroll=False)
    for grp in range(2):
        for k in range(GRP):
            drain(data[grp][k], gsem[grp])
    plsc.subcore_barrier()
    pltpu.sync_copy(acc.at[pl.ds(sid * RPT, RPT)],
                    out_hbm.at[cid, pl.ds(sid * RPT, RPT)])


def _sc_edge_pass(table, p_idx, zeros_h):
    return pl.kernel(
        _sc_edge_body,
        out_type=jax.ShapeDtypeStruct((NC, AROWS, H), jnp.float32),
        mesh=_sc_mesh(),
        compiler_params=pltpu.CompilerParams(use_tc_tiling_on_sc=False),
        scratch_types=[
            pltpu.VMEM((NCHUNK, CH), jnp.int32),
        ] + [pltpu.VMEM((CH,), jnp.int32) for _ in range(4 * GRP)]
          + [pltpu.VMEM((CH, H), jnp.float32) for _ in range(2 * GRP)] + [
            pltpu.VMEM_SHARED((AROWS, H), jnp.float32),
            pltpu.SemaphoreType.DMA,
            pltpu.SemaphoreType.DMA,
            pltpu.SemaphoreType.DMA,
            pltpu.SemaphoreType.DMA,
        ],
    )(table, p_idx, zeros_h)


def _sc_count_body(sidx_hbm, ones_hbm, zeros_hbm, out_hbm, sv, ones_v, acc):
    cid = lax.axis_index("c")
    sid = lax.axis_index("s")
    wid = cid * NS + sid
    pltpu.sync_copy(zeros_hbm, acc.at[pl.ds(sid * RPT, RPT)])
    plsc.subcore_barrier()
    pltpu.sync_copy(sidx_hbm.at[wid], sv)
    pltpu.sync_copy(ones_hbm, ones_v)

    def chunk(j, carry):
        pltpu.sync_copy(ones_v, acc.at[sv.at[j]], add=True)
        return carry

    lax.fori_loop(0, NCHUNK, chunk, 0, unroll=False)
    plsc.subcore_barrier()
    pltpu.sync_copy(acc.at[pl.ds(sid * RPT, RPT)],
                    out_hbm.at[cid, pl.ds(sid * RPT, RPT)])


def _sc_count_pass(s_idx, ones_c, zeros_h):
    return pl.kernel(
        _sc_count_body,
        out_type=jax.ShapeDtypeStruct((NC, AROWS, H), jnp.float32),
        mesh=_sc_mesh(),
        compiler_params=pltpu.CompilerParams(use_tc_tiling_on_sc=False),
        scratch_types=[
            pltpu.VMEM((NCHUNK, CH), jnp.int32),
            pltpu.VMEM((CH, H), jnp.float32),
            pltpu.VMEM_SHARED((AROWS, H), jnp.float32),
        ],
    )(s_idx, ones_c, zeros_h)


def _mm_body(h_ref, bases_ref, comp_ref, root_ref, bias_ref, y_ref, out0_ref):
    h = h_ref[...]
    w = jnp.tensordot(comp_ref[...], bases_ref[...], axes=[[1], [0]])
    for r in range(R):
        y_ref[pl.ds(r * N4, N4), :] = jnp.dot(
            h, w[r], preferred_element_type=jnp.float32)
    out0_ref[...] = (
        jnp.dot(h, root_ref[...], preferred_element_type=jnp.float32)
        + bias_ref[...])


def _mm_pass(h4, bases4, comp, root4, bias4):
    return pl.pallas_call(
        _mm_body,
        out_shape=[
            jax.ShapeDtypeStruct((R * N4, 128), jnp.float32),
            jax.ShapeDtypeStruct((N4, 128), jnp.float32),
        ],
    )(h4, bases4, comp, root4, bias4)


def _prep_body(c_ref, rw_ref):
    rw_ref[...] = 1.0 / jnp.maximum(c_ref[0] + c_ref[1], 1.0)


def _prep_pass(cnt2):
    return pl.pallas_call(
        _prep_body,
        out_shape=jax.ShapeDtypeStruct((A4, 128), jnp.float32),
    )(cnt2)


def _dg_lhsT(a, b):
    return lax.dot_general(a, b, (((0,), (0,)), ((), ())),
                           preferred_element_type=jnp.float32)


def _dg_rhsT(a, b):
    return lax.dot_general(a, b, (((1,), (1,)), ((), ())),
                           preferred_element_type=jnp.float32)


def _agg_norm_body(s_ref, rw_ref, out0_ref, g_ref, u_ref, k_ref,
                   w_ref, b_ref, ms_ref, y_ref, uit_ref):
    acc = out0_ref[...]
    for r in range(R):
        s = s_ref[0, pl.ds(r * N4, N4), :] + s_ref[1, pl.ds(r * N4, N4), :]
        acc = acc + s * rw_ref[pl.ds(r * N4, N4), :]
    gmat = g_ref[...]
    kmat = k_ref[...]
    t = jnp.dot(gmat, jnp.dot(acc, kmat, preferred_element_type=jnp.float32),
                preferred_element_type=jnp.float32) * (1.0 / PG)
    meanb = _dg_lhsT(gmat, _dg_rhsT(t, kmat))
    sub = acc - meanb * ms_ref[...]
    tv = jnp.dot(gmat, jnp.dot(sub * sub, kmat,
                               preferred_element_type=jnp.float32),
                 preferred_element_type=jnp.float32) * (1.0 / PG)
    varb = _dg_lhsT(gmat, _dg_rhsT(tv, kmat))
    yn = jnp.tanh(w_ref[...] * sub / jnp.sqrt(varb + 1e-6) + b_ref[...])
    y_ref[...] = yn
    uit_ref[...] = jnp.dot(u_ref[...], yn, preferred_element_type=jnp.float32)


def _agg_norm_pass(s2, rw4, out0, gmat, umat, kmat, w, b, ms):
    return pl.pallas_call(
        _agg_norm_body,
        out_shape=[
            jax.ShapeDtypeStruct((N4, 128), jnp.float32),
            jax.ShapeDtypeStruct((NG, 128), jnp.float32),
        ],
    )(s2, rw4, out0, gmat, umat, kmat,
      w.reshape(1, 128), b.reshape(1, 128), ms.reshape(1, 128))


def _mlp_body(u0, u1, u2, u3, w1_ref, b1_ref, w2_ref, b2_ref, o_ref):
    cols = []
    for ref in (u0, u1, u2, u3):
        cols.append(ref[:, 0:H])
    for ref in (u0, u1, u2, u3):
        cols.append(ref[:, H:2 * H])
    g = jnp.concatenate(cols, axis=1)
    h1 = jnp.maximum(
        jnp.dot(g, w1_ref[...], preferred_element_type=jnp.float32)
        + b1_ref[...], 0.0)
    logits = (jnp.dot(h1, w2_ref[...], preferred_element_type=jnp.float32)
              + b2_ref[...])
    m = jnp.max(logits, axis=-1, keepdims=True)
    z = logits - m
    o_ref[...] = z - jnp.log(jnp.sum(jnp.exp(z), axis=-1, keepdims=True))


def _mlp_pass(uits, w1, b1, w2, b2):
    n_cls = w2.shape[1]
    return pl.pallas_call(
        _mlp_body,
        out_shape=jax.ShapeDtypeStruct((NG, n_cls), jnp.float32),
    )(*uits, w1, b1.reshape(1, -1), w2, b2.reshape(1, -1))


def kernel(x, edge_index, edge_type, batch, conv0_bases, conv0_comp,
           conv0_root, conv0_bias, convs_bases, convs_comp, convs_root,
           convs_bias, norm_weight, norm_bias, norm_scale, lin1_w, lin1_b,
           lin2_w, lin2_b):
    src, dst = edge_index[0], edge_index[1]
    g_idx = edge_type * N + src
    s_idx = edge_type * N + dst
    pad = EPAD - E
    g_idx = jnp.concatenate([g_idx, jnp.zeros((pad,), jnp.int32)])
    s_idx = jnp.concatenate([s_idx, jnp.full((pad,), R * N, jnp.int32)])
    p_idx = (g_idx | (s_idx << 16)).reshape(NW, NCHUNK, CH)
    s_idx = s_idx.reshape(NW, NCHUNK, CH)

    zeros_h = jnp.zeros((RPT, H), jnp.float32)
    ones_c = jnp.ones((CH, H), jnp.float32)
    eye4 = jnp.eye(4, dtype=jnp.float32)
    kmat = jnp.tile(jnp.eye(H, dtype=jnp.float32), (4, 1))
    gmat = jnp.kron(jnp.eye(NG, dtype=jnp.float32),
                    jnp.ones((1, 5), jnp.float32))
    umat = jnp.kron(jnp.eye(NG, dtype=jnp.float32),
                    jnp.array([[1.0, 0, 0, 0, 0]], jnp.float32))

    cnt = _sc_count_pass(s_idx, ones_c, zeros_h)
    rw4 = _prep_pass(cnt.reshape(NC, A4, 128))

    uits = []
    h4 = x.reshape(N4, 4 * x.shape[1])
    for l in range(4):
        if l == 0:
            bases, comp, root, bias = (conv0_bases, conv0_comp, conv0_root,
                                       conv0_bias)
        else:
            bases, comp, root, bias = (convs_bases[l - 1], convs_comp[l - 1],
                                       convs_root[l - 1], convs_bias[l - 1])
        bases4 = jax.vmap(lambda bb: jnp.kron(eye4, bb))(bases)
        root4 = jnp.kron(eye4, root)
        bias4 = jnp.tile(bias, 4).reshape(1, 128)
        y4, out0 = _mm_pass(h4, bases4, comp, root4, bias4)
        s_acc = _sc_edge_pass(y4.reshape(R * N, H), p_idx, zeros_h)
        w128 = jnp.tile(norm_weight[l], 4)
        b128 = jnp.tile(norm_bias[l], 4)
        ms128 = jnp.tile(norm_scale[l], 4)
        h4, uit = _agg_norm_pass(s_acc.reshape(NC, A4, 128), rw4, out0,
                                 gmat, umat, kmat, w128, b128, ms128)
        uits.append(uit)

    return _mlp_pass(uits, lin1_w, lin1_b, lin2_w, lin2_b)


# speedup vs baseline: 1.0550x; 1.0550x over previous
import functools

import jax
import jax.numpy as jnp
from jax import lax
from jax.experimental import pallas as pl
from jax.experimental.pallas import tpu as pltpu
from jax.experimental.pallas import tpu_sc as plsc

N = 10000
E = 320000
R = 5
NG = 500
PG = 20
H = 32
NC, NS = 2, 16
NW = NC * NS
CH = 128
NCHUNK = 80
EPT = CH * NCHUNK
EPAD = EPT * NW
AROWS = R * N + 48
RPT = AROWS // NS
N4 = N // 4
A4 = AROWS // 4


@functools.lru_cache(maxsize=None)
def _sc_mesh():
    return plsc.VectorSubcoreMesh(
        core_axis_name="c", subcore_axis_name="s",
        num_cores=NC, num_subcores=NS)


def _sc_edge_body(table_hbm, gidx_hbm, sidx_hbm, zeros_hbm, out_hbm,
                  gv, sv, d0, d1, acc, semg0, semg1, sems0, sems1):
    cid = lax.axis_index("c")
    sid = lax.axis_index("s")
    wid = cid * NS + sid
    pltpu.sync_copy(zeros_hbm, acc.at[pl.ds(sid * RPT, RPT)])
    plsc.subcore_barrier()
    pltpu.sync_copy(gidx_hbm.at[wid], gv)
    pltpu.sync_copy(sidx_hbm.at[wid], sv)

    def drain(buf, sem):
        pltpu.make_async_copy(table_hbm.at[pl.ds(0, CH)], buf, sem).wait()

    pltpu.async_copy(table_hbm.at[gv.at[0]], d0, semg0)
    pltpu.async_copy(table_hbm.at[gv.at[1]], d1, semg1)

    def step(j2, carry):
        j = j2 * 2
        drain(d0, semg0)
        pltpu.async_copy(d0, acc.at[sv.at[j]], sems0, add=True)
        drain(d1, semg1)
        pltpu.async_copy(d1, acc.at[sv.at[j + 1]], sems1, add=True)
        drain(d0, sems0)
        pltpu.async_copy(table_hbm.at[gv.at[jnp.minimum(j + 2, NCHUNK - 1)]],
                         d0, semg0)
        drain(d1, sems1)
        pltpu.async_copy(table_hbm.at[gv.at[jnp.minimum(j + 3, NCHUNK - 1)]],
                         d1, semg1)
        return carry

    lax.fori_loop(0, NCHUNK // 2, step, 0, unroll=False)
    drain(d0, semg0)
    drain(d1, semg1)
    plsc.subcore_barrier()
    pltpu.sync_copy(acc.at[pl.ds(sid * RPT, RPT)],
                    out_hbm.at[cid, pl.ds(sid * RPT, RPT)])


def _sc_edge_pass(table, g_idx, s_idx, zeros_h):
    return pl.kernel(
        _sc_edge_body,
        out_type=jax.ShapeDtypeStruct((NC, AROWS, H), jnp.float32),
        mesh=_sc_mesh(),
        compiler_params=pltpu.CompilerParams(use_tc_tiling_on_sc=False),
        scratch_types=[
            pltpu.VMEM((NCHUNK, CH), jnp.int32),
            pltpu.VMEM((NCHUNK, CH), jnp.int32),
            pltpu.VMEM((CH, H), jnp.float32),
            pltpu.VMEM((CH, H), jnp.float32),
            pltpu.VMEM_SHARED((AROWS, H), jnp.float32),
            pltpu.SemaphoreType.DMA,
            pltpu.SemaphoreType.DMA,
            pltpu.SemaphoreType.DMA,
            pltpu.SemaphoreType.DMA,
        ],
    )(table, g_idx, s_idx, zeros_h)


def _sc_count_body(sidx_hbm, ones_hbm, zeros_hbm, out_hbm, sv, ones_v, acc):
    cid = lax.axis_index("c")
    sid = lax.axis_index("s")
    wid = cid * NS + sid
    pltpu.sync_copy(zeros_hbm, acc.at[pl.ds(sid * RPT, RPT)])
    plsc.subcore_barrier()
    pltpu.sync_copy(sidx_hbm.at[wid], sv)
    pltpu.sync_copy(ones_hbm, ones_v)

    def chunk(j, carry):
        pltpu.sync_copy(ones_v, acc.at[sv.at[j]], add=True)
        return carry

    lax.fori_loop(0, NCHUNK, chunk, 0, unroll=False)
    plsc.subcore_barrier()
    pltpu.sync_copy(acc.at[pl.ds(sid * RPT, RPT)],
                    out_hbm.at[cid, pl.ds(sid * RPT, RPT)])


def _sc_count_pass(s_idx, ones_c, zeros_h):
    return pl.kernel(
        _sc_count_body,
        out_type=jax.ShapeDtypeStruct((NC, AROWS, H), jnp.float32),
        mesh=_sc_mesh(),
        compiler_params=pltpu.CompilerParams(use_tc_tiling_on_sc=False),
        scratch_types=[
            pltpu.VMEM((NCHUNK, CH), jnp.int32),
            pltpu.VMEM((CH, H), jnp.float32),
            pltpu.VMEM_SHARED((AROWS, H), jnp.float32),
        ],
    )(s_idx, ones_c, zeros_h)


def _mm_body(h_ref, bases_ref, comp_ref, root_ref, bias_ref, y_ref, out0_ref):
    h = h_ref[...]
    w = jnp.tensordot(comp_ref[...], bases_ref[...], axes=[[1], [0]])
    for r in range(R):
        y_ref[pl.ds(r * N4, N4), :] = jnp.dot(
            h, w[r], preferred_element_type=jnp.float32)
    out0_ref[...] = (
        jnp.dot(h, root_ref[...], preferred_element_type=jnp.float32)
        + bias_ref[...])


def _mm_pass(h4, bases4, comp, root4, bias4):
    return pl.pallas_call(
        _mm_body,
        out_shape=[
            jax.ShapeDtypeStruct((R * N4, 128), jnp.float32),
            jax.ShapeDtypeStruct((N4, 128), jnp.float32),
        ],
    )(h4, bases4, comp, root4, bias4)


def _prep_body(c_ref, rw_ref):
    rw_ref[...] = 1.0 / jnp.maximum(c_ref[0] + c_ref[1], 1.0)


def _prep_pass(cnt2):
    return pl.pallas_call(
        _prep_body,
        out_shape=jax.ShapeDtypeStruct((A4, 128), jnp.float32),
    )(cnt2)


def _dg_lhsT(a, b):
    return lax.dot_general(a, b, (((0,), (0,)), ((), ())),
                           preferred_element_type=jnp.float32)


def _dg_rhsT(a, b):
    return lax.dot_general(a, b, (((1,), (1,)), ((), ())),
                           preferred_element_type=jnp.float32)


def _agg_norm_body(s_ref, rw_ref, out0_ref, g_ref, u_ref, k_ref,
                   w_ref, b_ref, ms_ref, y_ref, uit_ref):
    acc = out0_ref[...]
    for r in range(R):
        s = s_ref[0, pl.ds(r * N4, N4), :] + s_ref[1, pl.ds(r * N4, N4), :]
        acc = acc + s * rw_ref[pl.ds(r * N4, N4), :]
    gmat = g_ref[...]
    kmat = k_ref[...]
    t = jnp.dot(gmat, jnp.dot(acc, kmat, preferred_element_type=jnp.float32),
                preferred_element_type=jnp.float32) * (1.0 / PG)
    meanb = _dg_lhsT(gmat, _dg_rhsT(t, kmat))
    sub = acc - meanb * ms_ref[...]
    tv = jnp.dot(gmat, jnp.dot(sub * sub, kmat,
                               preferred_element_type=jnp.float32),
                 preferred_element_type=jnp.float32) * (1.0 / PG)
    varb = _dg_lhsT(gmat, _dg_rhsT(tv, kmat))
    yn = jnp.tanh(w_ref[...] * sub / jnp.sqrt(varb + 1e-6) + b_ref[...])
    y_ref[...] = yn
    uit_ref[...] = jnp.dot(u_ref[...], yn, preferred_element_type=jnp.float32)


def _agg_norm_pass(s2, rw4, out0, gmat, umat, kmat, w, b, ms):
    return pl.pallas_call(
        _agg_norm_body,
        out_shape=[
            jax.ShapeDtypeStruct((N4, 128), jnp.float32),
            jax.ShapeDtypeStruct((NG, 128), jnp.float32),
        ],
    )(s2, rw4, out0, gmat, umat, kmat,
      w.reshape(1, 128), b.reshape(1, 128), ms.reshape(1, 128))


def _mlp_body(u0, u1, u2, u3, w1_ref, b1_ref, w2_ref, b2_ref, o_ref):
    cols = []
    for ref in (u0, u1, u2, u3):
        cols.append(ref[:, 0:H])
    for ref in (u0, u1, u2, u3):
        cols.append(ref[:, H:2 * H])
    g = jnp.concatenate(cols, axis=1)
    h1 = jnp.maximum(
        jnp.dot(g, w1_ref[...], preferred_element_type=jnp.float32)
        + b1_ref[...], 0.0)
    logits = (jnp.dot(h1, w2_ref[...], preferred_element_type=jnp.float32)
              + b2_ref[...])
    m = jnp.max(logits, axis=-1, keepdims=True)
    z = logits - m
    o_ref[...] = z - jnp.log(jnp.sum(jnp.exp(z), axis=-1, keepdims=True))


def _mlp_pass(uits, w1, b1, w2, b2):
    n_cls = w2.shape[1]
    return pl.pallas_call(
        _mlp_body,
        out_shape=jax.ShapeDtypeStruct((NG, n_cls), jnp.float32),
    )(*uits, w1, b1.reshape(1, -1), w2, b2.reshape(1, -1))


def kernel(x, edge_index, edge_type, batch, conv0_bases, conv0_comp,
           conv0_root, conv0_bias, convs_bases, convs_comp, convs_root,
           convs_bias, norm_weight, norm_bias, norm_scale, lin1_w, lin1_b,
           lin2_w, lin2_b):
    src, dst = edge_index[0], edge_index[1]
    g_idx = edge_type * N + src
    s_idx = edge_type * N + dst
    pad = EPAD - E
    g_idx = jnp.concatenate([g_idx, jnp.zeros((pad,), jnp.int32)])
    s_idx = jnp.concatenate([s_idx, jnp.full((pad,), R * N, jnp.int32)])
    g_idx = g_idx.reshape(NW, NCHUNK, CH)
    s_idx = s_idx.reshape(NW, NCHUNK, CH)

    zeros_h = jnp.zeros((RPT, H), jnp.float32)
    ones_c = jnp.ones((CH, H), jnp.float32)
    eye4 = jnp.eye(4, dtype=jnp.float32)
    kmat = jnp.tile(jnp.eye(H, dtype=jnp.float32), (4, 1))
    gmat = jnp.kron(jnp.eye(NG, dtype=jnp.float32),
                    jnp.ones((1, 5), jnp.float32))
    umat = jnp.kron(jnp.eye(NG, dtype=jnp.float32),
                    jnp.array([[1.0, 0, 0, 0, 0]], jnp.float32))

    cnt = _sc_count_pass(s_idx, ones_c, zeros_h)
    rw4 = _prep_pass(cnt.reshape(NC, A4, 128))

    uits = []
    h4 = x.reshape(N4, 4 * x.shape[1])
    for l in range(4):
        if l == 0:
            bases, comp, root, bias = (conv0_bases, conv0_comp, conv0_root,
                                       conv0_bias)
        else:
            bases, comp, root, bias = (convs_bases[l - 1], convs_comp[l - 1],
                                       convs_root[l - 1], convs_bias[l - 1])
        bases4 = jax.vmap(lambda bb: jnp.kron(eye4, bb))(bases)
        root4 = jnp.kron(eye4, root)
        bias4 = jnp.tile(bias, 4).reshape(1, 128)
        y4, out0 = _mm_pass(h4, bases4, comp, root4, bias4)
        s_acc = _sc_edge_pass(y4.reshape(R * N, H), g_idx, s_idx, zeros_h)
        w128 = jnp.tile(norm_weight[l], 4)
        b128 = jnp.tile(norm_bias[l], 4)
        ms128 = jnp.tile(norm_scale[l], 4)
        h4, uit = _agg_norm_pass(s_acc.reshape(NC, A4, 128), rw4, out0,
                                 gmat, umat, kmat, w128, b128, ms128)
        uits.append(uit)

    return _mlp_pass(uits, lin1_w, lin1_b, lin2_w, lin2_b)


# speedup vs baseline: 1.1275x; 1.0687x over previous
import functools

import jax
import jax.numpy as jnp
from jax import lax
from jax.experimental import pallas as pl
from jax.experimental.pallas import tpu as pltpu
from jax.experimental.pallas import tpu_sc as plsc

N = 10000
E = 320000
R = 5
NG = 500
PG = 20
H = 32
NC, NS = 2, 16
NW = NC * NS
CH = 64
NCHUNK = 160
EPT = CH * NCHUNK
EPAD = EPT * NW
AROWS = R * N + 48
RPT = AROWS // NS
N4 = N // 4
A4 = AROWS // 4


@functools.lru_cache(maxsize=None)
def _sc_mesh():
    return plsc.VectorSubcoreMesh(
        core_axis_name="c", subcore_axis_name="s",
        num_cores=NC, num_subcores=NS)


def _sc_edge_body(table_hbm, gidx_hbm, sidx_hbm, zeros_hbm, out_hbm,
                  gv, sv, d0, d1, d2, d3, acc,
                  semg0, semg1, semg2, semg3):
    cid = lax.axis_index("c")
    sid = lax.axis_index("s")
    wid = cid * NS + sid
    pltpu.sync_copy(zeros_hbm, acc.at[pl.ds(sid * RPT, RPT)])
    plsc.subcore_barrier()
    pltpu.sync_copy(gidx_hbm.at[wid], gv)
    pltpu.sync_copy(sidx_hbm.at[wid], sv)

    bufs = [d0, d1, d2, d3]
    sems = [semg0, semg1, semg2, semg3]

    def drain(buf, sem):
        pltpu.make_async_copy(table_hbm.at[pl.ds(0, CH)], buf, sem).wait()

    for k in range(4):
        pltpu.async_copy(table_hbm.at[gv.at[k]], bufs[k], sems[k])

    def step(j4, carry):
        j = j4 * 4
        for k in range(4):
            drain(bufs[k], sems[k])
            pltpu.sync_copy(bufs[k], acc.at[sv.at[j + k]], add=True)
            jn = jnp.minimum(j + 4 + k, NCHUNK - 1)
            pltpu.async_copy(table_hbm.at[gv.at[jn]], bufs[k], sems[k])
        return carry

    lax.fori_loop(0, NCHUNK // 4, step, 0, unroll=False)
    for k in range(4):
        drain(bufs[k], sems[k])
    plsc.subcore_barrier()
    pltpu.sync_copy(acc.at[pl.ds(sid * RPT, RPT)],
                    out_hbm.at[cid, pl.ds(sid * RPT, RPT)])


def _sc_edge_pass(table, g_idx, s_idx, zeros_h):
    return pl.kernel(
        _sc_edge_body,
        out_type=jax.ShapeDtypeStruct((NC, AROWS, H), jnp.float32),
        mesh=_sc_mesh(),
        compiler_params=pltpu.CompilerParams(use_tc_tiling_on_sc=False),
        scratch_types=[
            pltpu.VMEM((NCHUNK, CH), jnp.int32),
            pltpu.VMEM((NCHUNK, CH), jnp.int32),
            pltpu.VMEM((CH, H), jnp.float32),
            pltpu.VMEM((CH, H), jnp.float32),
            pltpu.VMEM((CH, H), jnp.float32),
            pltpu.VMEM((CH, H), jnp.float32),
            pltpu.VMEM_SHARED((AROWS, H), jnp.float32),
            pltpu.SemaphoreType.DMA,
            pltpu.SemaphoreType.DMA,
            pltpu.SemaphoreType.DMA,
            pltpu.SemaphoreType.DMA,
        ],
    )(table, g_idx, s_idx, zeros_h)


def _sc_count_body(sidx_hbm, ones_hbm, zeros_hbm, out_hbm, sv, ones_v, acc,
                   sem):
    cid = lax.axis_index("c")
    sid = lax.axis_index("s")
    wid = cid * NS + sid
    pltpu.sync_copy(zeros_hbm, acc.at[pl.ds(sid * RPT, RPT)])
    plsc.subcore_barrier()
    pltpu.sync_copy(sidx_hbm.at[wid], sv)
    pltpu.sync_copy(ones_hbm, ones_v)

    def chunk(j, carry):
        pltpu.async_copy(ones_v, acc.at[sv.at[j]], sem, add=True)
        return carry

    lax.fori_loop(0, NCHUNK, chunk, 0, unroll=False)

    def drainc(j, carry):
        pltpu.make_async_copy(ones_hbm, ones_v, sem).wait()
        return carry

    lax.fori_loop(0, NCHUNK, drainc, 0, unroll=False)
    plsc.subcore_barrier()
    pltpu.sync_copy(acc.at[pl.ds(sid * RPT, RPT)],
                    out_hbm.at[cid, pl.ds(sid * RPT, RPT)])


def _sc_count_pass(s_idx, ones_c, zeros_h):
    return pl.kernel(
        _sc_count_body,
        out_type=jax.ShapeDtypeStruct((NC, AROWS, H), jnp.float32),
        mesh=_sc_mesh(),
        compiler_params=pltpu.CompilerParams(use_tc_tiling_on_sc=False),
        scratch_types=[
            pltpu.VMEM((NCHUNK, CH), jnp.int32),
            pltpu.VMEM((CH, H), jnp.float32),
            pltpu.VMEM_SHARED((AROWS, H), jnp.float32),
            pltpu.SemaphoreType.DMA,
        ],
    )(s_idx, ones_c, zeros_h)


def _mm_body(h_ref, bases_ref, comp_ref, root_ref, bias_ref, y_ref, out0_ref):
    h = h_ref[...]
    w = jnp.tensordot(comp_ref[...], bases_ref[...], axes=[[1], [0]])
    for r in range(R):
        y_ref[pl.ds(r * N4, N4), :] = jnp.dot(
            h, w[r], preferred_element_type=jnp.float32)
    out0_ref[...] = (
        jnp.dot(h, root_ref[...], preferred_element_type=jnp.float32)
        + bias_ref[...])


def _mm_pass(h4, bases4, comp, root4, bias4):
    return pl.pallas_call(
        _mm_body,
        out_shape=[
            jax.ShapeDtypeStruct((R * N4, 128), jnp.float32),
            jax.ShapeDtypeStruct((N4, 128), jnp.float32),
        ],
    )(h4, bases4, comp, root4, bias4)


def _prep_body(c_ref, rw_ref):
    rw_ref[...] = 1.0 / jnp.maximum(c_ref[0] + c_ref[1], 1.0)


def _prep_pass(cnt2):
    return pl.pallas_call(
        _prep_body,
        out_shape=jax.ShapeDtypeStruct((A4, 128), jnp.float32),
    )(cnt2)


def _dg_lhsT(a, b):
    return lax.dot_general(a, b, (((0,), (0,)), ((), ())),
                           preferred_element_type=jnp.float32)


def _dg_rhsT(a, b):
    return lax.dot_general(a, b, (((1,), (1,)), ((), ())),
                           preferred_element_type=jnp.float32)


def _agg_norm_body(s_ref, rw_ref, out0_ref, g_ref, u_ref, k_ref,
                   w_ref, b_ref, ms_ref, y_ref, uit_ref):
    acc = out0_ref[...]
    for r in range(R):
        s = s_ref[0, pl.ds(r * N4, N4), :] + s_ref[1, pl.ds(r * N4, N4), :]
        acc = acc + s * rw_ref[pl.ds(r * N4, N4), :]
    gmat = g_ref[...]
    kmat = k_ref[...]
    t = jnp.dot(gmat, jnp.dot(acc, kmat, preferred_element_type=jnp.float32),
                preferred_element_type=jnp.float32) * (1.0 / PG)
    meanb = _dg_lhsT(gmat, _dg_rhsT(t, kmat))
    sub = acc - meanb * ms_ref[...]
    tv = jnp.dot(gmat, jnp.dot(sub * sub, kmat,
                               preferred_element_type=jnp.float32),
                 preferred_element_type=jnp.float32) * (1.0 / PG)
    varb = _dg_lhsT(gmat, _dg_rhsT(tv, kmat))
    yn = jnp.tanh(w_ref[...] * sub / jnp.sqrt(varb + 1e-6) + b_ref[...])
    y_ref[...] = yn
    uit_ref[...] = jnp.dot(u_ref[...], yn, preferred_element_type=jnp.float32)


def _agg_norm_pass(s2, rw4, out0, gmat, umat, kmat, w, b, ms):
    return pl.pallas_call(
        _agg_norm_body,
        out_shape=[
            jax.ShapeDtypeStruct((N4, 128), jnp.float32),
            jax.ShapeDtypeStruct((NG, 128), jnp.float32),
        ],
    )(s2, rw4, out0, gmat, umat, kmat,
      w.reshape(1, 128), b.reshape(1, 128), ms.reshape(1, 128))


def _mlp_body(u0, u1, u2, u3, w1_ref, b1_ref, w2_ref, b2_ref, o_ref):
    cols = []
    for ref in (u0, u1, u2, u3):
        cols.append(ref[:, 0:H])
    for ref in (u0, u1, u2, u3):
        cols.append(ref[:, H:2 * H])
    g = jnp.concatenate(cols, axis=1)
    h1 = jnp.maximum(
        jnp.dot(g, w1_ref[...], preferred_element_type=jnp.float32)
        + b1_ref[...], 0.0)
    logits = (jnp.dot(h1, w2_ref[...], preferred_element_type=jnp.float32)
              + b2_ref[...])
    m = jnp.max(logits, axis=-1, keepdims=True)
    z = logits - m
    o_ref[...] = z - jnp.log(jnp.sum(jnp.exp(z), axis=-1, keepdims=True))


def _mlp_pass(uits, w1, b1, w2, b2):
    n_cls = w2.shape[1]
    return pl.pallas_call(
        _mlp_body,
        out_shape=jax.ShapeDtypeStruct((NG, n_cls), jnp.float32),
    )(*uits, w1, b1.reshape(1, -1), w2, b2.reshape(1, -1))


def kernel(x, edge_index, edge_type, batch, conv0_bases, conv0_comp,
           conv0_root, conv0_bias, convs_bases, convs_comp, convs_root,
           convs_bias, norm_weight, norm_bias, norm_scale, lin1_w, lin1_b,
           lin2_w, lin2_b):
    src, dst = edge_index[0], edge_index[1]
    g_idx = edge_type * N + src
    s_idx = edge_type * N + dst
    pad = EPAD - E
    g_idx = jnp.concatenate([g_idx, jnp.zeros((pad,), jnp.int32)])
    s_idx = jnp.concatenate([s_idx, jnp.full((pad,), R * N, jnp.int32)])
    g_idx = g_idx.reshape(NW, NCHUNK, CH)
    s_idx = s_idx.reshape(NW, NCHUNK, CH)

    zeros_h = jnp.zeros((RPT, H), jnp.float32)
    ones_c = jnp.ones((CH, H), jnp.float32)
    eye4 = jnp.eye(4, dtype=jnp.float32)
    kmat = jnp.tile(jnp.eye(H, dtype=jnp.float32), (4, 1))
    gmat = jnp.kron(jnp.eye(NG, dtype=jnp.float32),
                    jnp.ones((1, 5), jnp.float32))
    umat = jnp.kron(jnp.eye(NG, dtype=jnp.float32),
                    jnp.array([[1.0, 0, 0, 0, 0]], jnp.float32))

    cnt = _sc_count_pass(s_idx, ones_c, zeros_h)
    rw4 = _prep_pass(cnt.reshape(NC, A4, 128))

    uits = []
    h4 = x.reshape(N4, 4 * x.shape[1])
    for l in range(4):
        if l == 0:
            bases, comp, root, bias = (conv0_bases, conv0_comp, conv0_root,
                                       conv0_bias)
        else:
            bases, comp, root, bias = (convs_bases[l - 1], convs_comp[l - 1],
                                       convs_root[l - 1], convs_bias[l - 1])
        bases4 = jax.vmap(lambda bb: jnp.kron(eye4, bb))(bases)
        root4 = jnp.kron(eye4, root)
        bias4 = jnp.tile(bias, 4).reshape(1, 128)
        y4, out0 = _mm_pass(h4, bases4, comp, root4, bias4)
        s_acc = _sc_edge_pass(y4.reshape(R * N, H), g_idx, s_idx, zeros_h)
        w128 = jnp.tile(norm_weight[l], 4)
        b128 = jnp.tile(norm_bias[l], 4)
        ms128 = jnp.tile(norm_scale[l], 4)
        h4, uit = _agg_norm_pass(s_acc.reshape(NC, A4, 128), rw4, out0,
                                 gmat, umat, kmat, w128, b128, ms128)
        uits.append(uit)

    return _mlp_pass(uits, lin1_w, lin1_b, lin2_w, lin2_b)


# speedup vs baseline: 1.1692x; 1.0369x over previous
import functools

import jax
import jax.numpy as jnp
from jax import lax
from jax.experimental import pallas as pl
from jax.experimental.pallas import tpu as pltpu
from jax.experimental.pallas import tpu_sc as plsc

N = 10000
E = 320000
R = 5
NG = 500
PG = 20
H = 32
NC, NS = 2, 16
NW = NC * NS
CH = 128
NCHUNK = 80
EPT = CH * NCHUNK
EPAD = EPT * NW
AROWS = R * N + 48
RPT = AROWS // NS
N4 = N // 4
A4 = AROWS // 4


@functools.lru_cache(maxsize=None)
def _sc_mesh():
    return plsc.VectorSubcoreMesh(
        core_axis_name="c", subcore_axis_name="s",
        num_cores=NC, num_subcores=NS)


def _sc_edge_body(table_hbm, gidx_hbm, sidx_hbm, zeros_hbm, out_hbm,
                  gv, sv, d0, d1, acc, semg0, semg1):
    cid = lax.axis_index("c")
    sid = lax.axis_index("s")
    wid = cid * NS + sid
    pltpu.sync_copy(zeros_hbm, acc.at[pl.ds(sid * RPT, RPT)])
    plsc.subcore_barrier()
    pltpu.sync_copy(gidx_hbm.at[wid], gv)
    pltpu.sync_copy(sidx_hbm.at[wid], sv)

    bufs = [d0, d1]
    sems = [semg0, semg1]

    def drain(buf, sem):
        pltpu.make_async_copy(table_hbm.at[pl.ds(0, CH)], buf, sem).wait()

    for k in range(2):
        pltpu.async_copy(table_hbm.at[gv.at[k]], bufs[k], sems[k])

    def step(j2, carry):
        j = j2 * 2
        for k in range(2):
            drain(bufs[k], sems[k])
            pltpu.sync_copy(bufs[k], acc.at[sv.at[j + k]], add=True)
            jn = jnp.minimum(j + 2 + k, NCHUNK - 1)
            pltpu.async_copy(table_hbm.at[gv.at[jn]], bufs[k], sems[k])
        return carry

    lax.fori_loop(0, NCHUNK // 2, step, 0, unroll=False)
    for k in range(2):
        drain(bufs[k], sems[k])
    plsc.subcore_barrier()
    pltpu.sync_copy(acc.at[pl.ds(sid * RPT, RPT)],
                    out_hbm.at[cid, pl.ds(sid * RPT, RPT)])


def _sc_edge_pass(table, g_idx, s_idx, zeros_h):
    return pl.kernel(
        _sc_edge_body,
        out_type=jax.ShapeDtypeStruct((NC, AROWS, H), jnp.float32),
        mesh=_sc_mesh(),
        compiler_params=pltpu.CompilerParams(use_tc_tiling_on_sc=False),
        scratch_types=[
            pltpu.VMEM((NCHUNK, CH), jnp.int32),
            pltpu.VMEM((NCHUNK, CH), jnp.int32),
            pltpu.VMEM((CH, H), jnp.float32),
            pltpu.VMEM((CH, H), jnp.float32),
            pltpu.VMEM_SHARED((AROWS, H), jnp.float32),
            pltpu.SemaphoreType.DMA,
            pltpu.SemaphoreType.DMA,
        ],
    )(table, g_idx, s_idx, zeros_h)


def _sc_count_body(sidx_hbm, ones_hbm, zeros_hbm, out_hbm, sv, ones_v, acc,
                   sem):
    cid = lax.axis_index("c")
    sid = lax.axis_index("s")
    wid = cid * NS + sid
    pltpu.sync_copy(zeros_hbm, acc.at[pl.ds(sid * RPT, RPT)])
    plsc.subcore_barrier()
    pltpu.sync_copy(sidx_hbm.at[wid], sv)
    pltpu.sync_copy(ones_hbm, ones_v)

    def chunk(j, carry):
        pltpu.async_copy(ones_v, acc.at[sv.at[j]], sem, add=True)
        return carry

    lax.fori_loop(0, NCHUNK, chunk, 0, unroll=False)

    def drainc(j, carry):
        pltpu.make_async_copy(ones_hbm, ones_v, sem).wait()
        return carry

    lax.fori_loop(0, NCHUNK, drainc, 0, unroll=False)
    plsc.subcore_barrier()
    pltpu.sync_copy(acc.at[pl.ds(sid * RPT, RPT)],
                    out_hbm.at[cid, pl.ds(sid * RPT, RPT)])


def _sc_count_pass(s_idx, ones_c, zeros_h):
    return pl.kernel(
        _sc_count_body,
        out_type=jax.ShapeDtypeStruct((NC, AROWS, H), jnp.float32),
        mesh=_sc_mesh(),
        compiler_params=pltpu.CompilerParams(use_tc_tiling_on_sc=False),
        scratch_types=[
            pltpu.VMEM((NCHUNK, CH), jnp.int32),
            pltpu.VMEM((CH, H), jnp.float32),
            pltpu.VMEM_SHARED((AROWS, H), jnp.float32),
            pltpu.SemaphoreType.DMA,
        ],
    )(s_idx, ones_c, zeros_h)


def _mm_body(h_ref, bases_ref, comp_ref, root_ref, bias_ref, y_ref, out0_ref):
    h = h_ref[...]
    w = jnp.tensordot(comp_ref[...], bases_ref[...], axes=[[1], [0]])
    for r in range(R):
        y_ref[pl.ds(r * N4, N4), :] = jnp.dot(
            h, w[r], preferred_element_type=jnp.float32)
    out0_ref[...] = (
        jnp.dot(h, root_ref[...], preferred_element_type=jnp.float32)
        + bias_ref[...])


def _mm_pass(h4, bases4, comp, root4, bias4):
    return pl.pallas_call(
        _mm_body,
        out_shape=[
            jax.ShapeDtypeStruct((R * N4, 128), jnp.float32),
            jax.ShapeDtypeStruct((N4, 128), jnp.float32),
        ],
    )(h4, bases4, comp, root4, bias4)


def _prep_body(c_ref, rw_ref):
    rw_ref[...] = 1.0 / jnp.maximum(c_ref[0] + c_ref[1], 1.0)


def _prep_pass(cnt2):
    return pl.pallas_call(
        _prep_body,
        out_shape=jax.ShapeDtypeStruct((A4, 128), jnp.float32),
    )(cnt2)


def _dg_lhsT(a, b):
    return lax.dot_general(a, b, (((0,), (0,)), ((), ())),
                           preferred_element_type=jnp.float32)


def _dg_rhsT(a, b):
    return lax.dot_general(a, b, (((1,), (1,)), ((), ())),
                           preferred_element_type=jnp.float32)


def _norm_core(s_ref, rw_ref, out0_ref, g_ref, u_ref, k_ref,
               w_ref, b_ref, ms_ref, uit_ref):
    acc = out0_ref[...]
    for r in range(R):
        s = s_ref[0, pl.ds(r * N4, N4), :] + s_ref[1, pl.ds(r * N4, N4), :]
        acc = acc + s * rw_ref[pl.ds(r * N4, N4), :]
    gmat = g_ref[...]
    kmat = k_ref[...]
    t = jnp.dot(gmat, jnp.dot(acc, kmat, preferred_element_type=jnp.float32),
                preferred_element_type=jnp.float32) * (1.0 / PG)
    meanb = _dg_lhsT(gmat, _dg_rhsT(t, kmat))
    sub = acc - meanb * ms_ref[...]
    tv = jnp.dot(gmat, jnp.dot(sub * sub, kmat,
                               preferred_element_type=jnp.float32),
                 preferred_element_type=jnp.float32) * (1.0 / PG)
    varb = _dg_lhsT(gmat, _dg_rhsT(tv, kmat))
    yn = jnp.tanh(w_ref[...] * sub / jnp.sqrt(varb + 1e-6) + b_ref[...])
    uit_ref[...] = jnp.dot(u_ref[...], yn, preferred_element_type=jnp.float32)
    return yn


def _agg_norm_body(s_ref, rw_ref, out0_ref, g_ref, u_ref, k_ref,
                   w_ref, b_ref, ms_ref, uit_ref):
    _norm_core(s_ref, rw_ref, out0_ref, g_ref, u_ref, k_ref,
               w_ref, b_ref, ms_ref, uit_ref)


def _agg_norm_pass(s2, rw4, out0, gmat, umat, kmat, w, b, ms):
    return pl.pallas_call(
        _agg_norm_body,
        out_shape=jax.ShapeDtypeStruct((NG, 128), jnp.float32),
    )(s2, rw4, out0, gmat, umat, kmat,
      w.reshape(1, 128), b.reshape(1, 128), ms.reshape(1, 128))


def _agg_mm_body(s_ref, rw_ref, out0_ref, g_ref, u_ref, k_ref,
                 w_ref, b_ref, ms_ref, bases_ref, comp_ref, root_ref,
                 bias_ref, uit_ref, y_ref, out0n_ref):
    yn = _norm_core(s_ref, rw_ref, out0_ref, g_ref, u_ref, k_ref,
                    w_ref, b_ref, ms_ref, uit_ref)
    wmat = jnp.tensordot(comp_ref[...], bases_ref[...], axes=[[1], [0]])
    for r in range(R):
        y_ref[pl.ds(r * N4, N4), :] = jnp.dot(
            yn, wmat[r], preferred_element_type=jnp.float32)
    out0n_ref[...] = (
        jnp.dot(yn, root_ref[...], preferred_element_type=jnp.float32)
        + bias_ref[...])


def _agg_mm_pass(s2, rw4, out0, gmat, umat, kmat, w, b, ms,
                 bases4, comp, root4, bias4):
    return pl.pallas_call(
        _agg_mm_body,
        out_shape=[
            jax.ShapeDtypeStruct((NG, 128), jnp.float32),
            jax.ShapeDtypeStruct((R * N4, 128), jnp.float32),
            jax.ShapeDtypeStruct((N4, 128), jnp.float32),
        ],
    )(s2, rw4, out0, gmat, umat, kmat,
      w.reshape(1, 128), b.reshape(1, 128), ms.reshape(1, 128),
      bases4, comp, root4, bias4)


def _mlp_body(u0, u1, u2, u3, w1_ref, b1_ref, w2_ref, b2_ref, o_ref):
    cols = []
    for ref in (u0, u1, u2, u3):
        cols.append(ref[:, 0:H])
    for ref in (u0, u1, u2, u3):
        cols.append(ref[:, H:2 * H])
    g = jnp.concatenate(cols, axis=1)
    h1 = jnp.maximum(
        jnp.dot(g, w1_ref[...], preferred_element_type=jnp.float32)
        + b1_ref[...], 0.0)
    logits = (jnp.dot(h1, w2_ref[...], preferred_element_type=jnp.float32)
              + b2_ref[...])
    m = jnp.max(logits, axis=-1, keepdims=True)
    z = logits - m
    o_ref[...] = z - jnp.log(jnp.sum(jnp.exp(z), axis=-1, keepdims=True))


def _mlp_pass(uits, w1, b1, w2, b2):
    n_cls = w2.shape[1]
    return pl.pallas_call(
        _mlp_body,
        out_shape=jax.ShapeDtypeStruct((NG, n_cls), jnp.float32),
    )(*uits, w1, b1.reshape(1, -1), w2, b2.reshape(1, -1))


def kernel(x, edge_index, edge_type, batch, conv0_bases, conv0_comp,
           conv0_root, conv0_bias, convs_bases, convs_comp, convs_root,
           convs_bias, norm_weight, norm_bias, norm_scale, lin1_w, lin1_b,
           lin2_w, lin2_b):
    src, dst = edge_index[0], edge_index[1]
    g_idx = edge_type * N + src
    s_idx = edge_type * N + dst
    pad = EPAD - E
    g_idx = jnp.concatenate([g_idx, jnp.zeros((pad,), jnp.int32)])
    s_idx = jnp.concatenate([s_idx, jnp.full((pad,), R * N, jnp.int32)])
    g_idx = g_idx.reshape(NW, NCHUNK, CH)
    s_idx = s_idx.reshape(NW, NCHUNK, CH)

    zeros_h = jnp.zeros((RPT, H), jnp.float32)
    ones_c = jnp.ones((CH, H), jnp.float32)
    eye4 = jnp.eye(4, dtype=jnp.float32)
    kmat = jnp.tile(jnp.eye(H, dtype=jnp.float32), (4, 1))
    gmat = jnp.kron(jnp.eye(NG, dtype=jnp.float32),
                    jnp.ones((1, 5), jnp.float32))
    umat = jnp.kron(jnp.eye(NG, dtype=jnp.float32),
                    jnp.array([[1.0, 0, 0, 0, 0]], jnp.float32))

    cnt = _sc_count_pass(s_idx, ones_c, zeros_h)
    rw4 = _prep_pass(cnt.reshape(NC, A4, 128))

    layer_w = []
    for l in range(4):
        if l == 0:
            bases, comp, root, bias = (conv0_bases, conv0_comp, conv0_root,
                                       conv0_bias)
        else:
            bases, comp, root, bias = (convs_bases[l - 1], convs_comp[l - 1],
                                       convs_root[l - 1], convs_bias[l - 1])
        layer_w.append((jax.vmap(lambda bb: jnp.kron(eye4, bb))(bases), comp,
                        jnp.kron(eye4, root), jnp.tile(bias, 4).reshape(1, 128)))

    uits = []
    x4 = x.reshape(N4, 4 * x.shape[1])
    y4, out0 = _mm_pass(x4, *layer_w[0])
    for l in range(4):
        s_acc = _sc_edge_pass(y4.reshape(R * N, H), g_idx, s_idx, zeros_h)
        args = (s_acc.reshape(NC, A4, 128), rw4, out0, gmat, umat, kmat,
                jnp.tile(norm_weight[l], 4), jnp.tile(norm_bias[l], 4),
                jnp.tile(norm_scale[l], 4))
        if l < 3:
            uit, y4, out0 = _agg_mm_pass(*args, *layer_w[l + 1])
        else:
            uit = _agg_norm_pass(*args)
        uits.append(uit)

    return _mlp_pass(uits, lin1_w, lin1_b, lin2_w, lin2_b)
